# baseline (speedup 1.0000x reference)
import jax
import jax.numpy as jnp
from jax import lax
from jax.experimental import pallas as pl
from jax.experimental.pallas import tpu as pltpu
from jax.experimental.pallas import tpu_sc as plsc

TABLE = 1_000_000
HID = 32
COL = 16
NC, NS = 2, 16
NW = NC * NS

_RB = 1000
_HALF = TABLE // 2


_LBLK = 16384
_SUB = _LBLK // 8
_GRID = -(-TABLE // _LBLK)
_NPACK = _GRID * _SUB


def _proj_body(embt_ref, w_ref, b_ref, out_ref):
    d = lax.dot_general(
        embt_ref[...], w_ref[...],
        (((0,), (0,)), ((), ())),
        preferred_element_type=jnp.float32,
    ) + b_ref[...]
    parts = [d[a * _SUB:(a + 1) * _SUB, :] for a in range(8)]
    out_ref[...] = jnp.concatenate(parts, axis=1)


def _project(embt, w, b):
    return pl.pallas_call(
        _proj_body,
        grid=(_GRID,),
        in_specs=[
            pl.BlockSpec((HID, _LBLK), lambda i: (0, i)),
            pl.BlockSpec((HID, COL), lambda i: (0, 0)),
            pl.BlockSpec((1, COL), lambda i: (0, 0)),
        ],
        out_specs=pl.BlockSpec((_SUB, 8 * COL), lambda i: (i, 0)),
        out_shape=jax.ShapeDtypeStruct((_NPACK, 8 * COL), jnp.float32),
    )(embt, w, b.reshape(1, COL))


_B = 819200
_HIST = 50
_BATCH = 16384
_IPR = 128
_ROWS = _B // _IPR
_BPW = _BATCH // NW
_BB = 64
_NB = _BPW // _BB
_JPB = _BB * _HIST // _IPR
_P = _HIST * COL


def _gather_body(table_hbm, idx_hbm, out_hbm, idx_v, rows_v, tbuf, gsem):
    wid = lax.axis_index("s") * NC + lax.axis_index("c")
    iot50 = jax.lax.broadcasted_iota(jnp.int32, (16,), 0) * _HIST

    def burst(g, _):
        pltpu.sync_copy(idx_hbm.at[pl.ds(wid * _NB * _JPB + g * _JPB, _JPB)],
                        idx_v)
        copies = []
        for j in range(_JPB):
            copies.append(
                pltpu.async_copy(
                    table_hbm.at[idx_v.at[j]],
                    rows_v.at[pl.ds(j * _IPR, _IPR)],
                    gsem,
                )
            )
        for c in copies:
            c.wait()

        def tp(h, _):
            ridx = [iot50 + (q * 16 * _HIST + h) for q in range(_BB // 16)]
            base = h * COL
            for cc in range(COL):
                cvec = jnp.full((16,), cc, dtype=jnp.int32)
                for q in range(_BB // 16):
                    tbuf[base + cc, pl.ds(q * 16, 16)] = plsc.load_gather(
                        rows_v, [ridx[q], cvec]
                    )
            return 0

        lax.fori_loop(0, _HIST, tp, 0)
        b0 = wid * _BPW + g * _BB
        pltpu.sync_copy(tbuf, out_hbm.at[:, pl.ds(b0, _BB)])
        return 0

    lax.fori_loop(0, _NB, burst, 0)


def _gather(table, idx2d):
    mesh = plsc.VectorSubcoreMesh(
        core_axis_name="c", subcore_axis_name="s", num_cores=NC, num_subcores=NS
    )
    f = pl.kernel(
        _gather_body,
        out_type=jax.ShapeDtypeStruct((_P, _BATCH), jnp.float32),
        mesh=mesh,
        scratch_types=[
            pltpu.VMEM((_JPB, _IPR), jnp.int32),
            pltpu.VMEM((_BB * _HIST, COL), jnp.float32),
            pltpu.VMEM((_P, _BB), jnp.float32),
            pltpu.SemaphoreType.DMA,
        ],
        compiler_params=pltpu.CompilerParams(
            use_tc_tiling_on_sc=False, needs_layout_passes=False
        ),
    )
    return f(table, idx2d)


def kernel(msg, emb_table, W_color, b_color):
    proj_packed = _project(emb_table.T, W_color, b_color)
    table = proj_packed.reshape(_NPACK * 8, COL)

    t = msg
    u = t & (_LBLK - 1)
    q = (t & ~(_LBLK - 1)) | ((u & (_SUB - 1)) << 3) | (u >> 11)

    out2 = _gather(table, q.reshape(_ROWS, _IPR))
    return jnp.transpose(out2.reshape(_HIST, COL, _BATCH), (2, 0, 1))


# speedup vs baseline: 1.0188x; 1.0188x over previous
import jax
import jax.numpy as jnp
from jax import lax
from jax.experimental import pallas as pl
from jax.experimental.pallas import tpu as pltpu
from jax.experimental.pallas import tpu_sc as plsc

TABLE = 1_000_000
HID = 32
COL = 16
NC, NS = 2, 16
NW = NC * NS

_RB = 1000
_HALF = TABLE // 2


_LBLK = 16384
_SUB = _LBLK // 8
_GRID = -(-TABLE // _LBLK)
_NPACK = _GRID * _SUB


def _proj_body(embt_ref, w_ref, b_ref, out_ref):
    d = lax.dot_general(
        embt_ref[...], w_ref[...],
        (((0,), (0,)), ((), ())),
        preferred_element_type=jnp.float32,
    )
    parts = [d[a * _SUB:(a + 1) * _SUB, :] for a in range(8)]
    out_ref[...] = jnp.concatenate(parts, axis=1) + b_ref[...]


def _project(embt, w, b):
    return pl.pallas_call(
        _proj_body,
        grid=(_GRID,),
        in_specs=[
            pl.BlockSpec((HID, _LBLK), lambda i: (0, i)),
            pl.BlockSpec((HID, COL), lambda i: (0, 0)),
            pl.BlockSpec((1, 8 * COL), lambda i: (0, 0)),
        ],
        out_specs=pl.BlockSpec((_SUB, 8 * COL), lambda i: (i, 0)),
        out_shape=jax.ShapeDtypeStruct((_NPACK, 8 * COL), jnp.float32),
    )(embt, w, jnp.tile(b, 8).reshape(1, 8 * COL))


_B = 819200
_HIST = 50
_BATCH = 16384
_IPR = 128
_ROWS = _B // _IPR
_BPW = _BATCH // NW
_BB = 64
_NB = _BPW // _BB
_JPB = _BB * _HIST // _IPR
_P = _HIST * COL


_HA = 25
_PA = _HA * COL


def _gather_body(table_hbm, idx_hbm, out_hbm, idx_v, rows_v, ta, tb,
                 gsem, wsa, wsb):
    wid = lax.axis_index("s") * NC + lax.axis_index("c")
    iot50 = jax.lax.broadcasted_iota(jnp.int32, (16,), 0) * _HIST

    def tp(buf, h0):
        def body(h, _):
            ridx = [iot50 + (q * 16 * _HIST + h0 + h) for q in range(_BB // 16)]
            base = h * COL
            for cc in range(COL):
                cvec = jnp.full((16,), cc, dtype=jnp.int32)
                for q in range(_BB // 16):
                    buf[base + cc, pl.ds(q * 16, 16)] = plsc.load_gather(
                        rows_v, [ridx[q], cvec]
                    )
            return 0

        lax.fori_loop(0, _HA, body, 0)

    def burst(g, _):
        pltpu.sync_copy(idx_hbm.at[pl.ds(wid * _NB * _JPB + g * _JPB, _JPB)],
                        idx_v)
        copies = []
        for j in range(_JPB):
            copies.append(
                pltpu.async_copy(
                    table_hbm.at[idx_v.at[j]],
                    rows_v.at[pl.ds(j * _IPR, _IPR)],
                    gsem,
                )
            )
        for c in copies:
            c.wait()

        b0 = wid * _BPW + g * _BB
        tp(ta, 0)
        @pl.when(g > 0)
        def _():
            pltpu.make_async_copy(
                tb, out_hbm.at[pl.ds(_PA, _PA), pl.ds(b0, _BB)], wsb
            ).wait()

        da = pltpu.async_copy(
            ta, out_hbm.at[pl.ds(0, _PA), pl.ds(b0, _BB)], wsa
        )
        tp(tb, _HA)
        pltpu.async_copy(
            tb, out_hbm.at[pl.ds(_PA, _PA), pl.ds(b0, _BB)], wsb
        )
        da.wait()
        return 0

    lax.fori_loop(0, _NB, burst, 0)
    bl = wid * _BPW + (_NB - 1) * _BB
    pltpu.make_async_copy(
        tb, out_hbm.at[pl.ds(_PA, _PA), pl.ds(bl, _BB)], wsb
    ).wait()


def _gather(table, idx2d):
    mesh = plsc.VectorSubcoreMesh(
        core_axis_name="c", subcore_axis_name="s", num_cores=NC, num_subcores=NS
    )
    f = pl.kernel(
        _gather_body,
        out_type=jax.ShapeDtypeStruct((_P, _BATCH), jnp.float32),
        mesh=mesh,
        scratch_types=[
            pltpu.VMEM((_JPB, _IPR), jnp.int32),
            pltpu.VMEM((_BB * _HIST, COL), jnp.float32),
            pltpu.VMEM((_PA, _BB), jnp.float32),
            pltpu.VMEM((_PA, _BB), jnp.float32),
            pltpu.SemaphoreType.DMA,
            pltpu.SemaphoreType.DMA,
            pltpu.SemaphoreType.DMA,
        ],
        compiler_params=pltpu.CompilerParams(
            use_tc_tiling_on_sc=False, needs_layout_passes=False
        ),
    )
    return f(table, idx2d)


def kernel(msg, emb_table, W_color, b_color):
    proj_packed = _project(emb_table.T, W_color, b_color)
    table = proj_packed.reshape(_NPACK * 8, COL)

    t = msg
    u = t & (_LBLK - 1)
    q = (t & ~(_LBLK - 1)) | ((u & (_SUB - 1)) << 3) | (u >> 11)

    out2 = _gather(table, q.reshape(_ROWS, _IPR))
    return jnp.transpose(out2.reshape(_HIST, COL, _BATCH), (2, 0, 1))
